# Initial kernel scaffold; baseline (speedup 1.0000x reference)
import jax
import jax.numpy as jnp
from jax.experimental import pallas as pl


def kernel(t, Vt, edge_index, in_graph_features, skip0, skip1, params):
    raise NotImplementedError("write your pallas kernel here")


# speedup vs baseline: 1.0055x; 1.0055x over previous
import functools

import jax
import jax.numpy as jnp
from jax.experimental import pallas as pl


def _dense_body(x_ref, agg_ref, w0_ref, w1_ref, b_ref, o_ref, *, relu):
    acc = x_ref[...] @ w0_ref[...] + agg_ref[...] @ w1_ref[...] + b_ref[...]
    if relu:
        acc = jnp.maximum(acc, 0.0)
    o_ref[...] = acc


def _dense_layer(x, agg, W0, W1, b, relu):
    n, d = x.shape
    c_out = W0.shape[1]
    blk = 2000
    grid = (n // blk,)
    return pl.pallas_call(
        functools.partial(_dense_body, relu=relu),
        grid=grid,
        in_specs=[
            pl.BlockSpec((blk, d), lambda i: (i, 0)),
            pl.BlockSpec((blk, d), lambda i: (i, 0)),
            pl.BlockSpec((d, c_out), lambda i: (0, 0)),
            pl.BlockSpec((d, c_out), lambda i: (0, 0)),
            pl.BlockSpec((1, c_out), lambda i: (0, 0)),
        ],
        out_specs=pl.BlockSpec((blk, c_out), lambda i: (i, 0)),
        out_shape=jax.ShapeDtypeStruct((n, c_out), jnp.float32),
    )(x, agg, W0, W1, b.reshape(1, -1))


def _trilinear(feat, coords):
    C, Dd, Hh, Ww = feat.shape
    x = (coords[:, 0] + 1.0) * 0.5 * (Ww - 1)
    y = (coords[:, 1] + 1.0) * 0.5 * (Hh - 1)
    z = (coords[:, 2] + 1.0) * 0.5 * (Dd - 1)
    x0 = jnp.floor(x); y0 = jnp.floor(y); z0 = jnp.floor(z)
    xd = (x - x0)[None, :]; yd = (y - y0)[None, :]; zd = (z - z0)[None, :]
    x0i = jnp.clip(x0.astype(jnp.int32), 0, Ww - 1); x1i = jnp.clip(x0i + 1, 0, Ww - 1)
    y0i = jnp.clip(y0.astype(jnp.int32), 0, Hh - 1); y1i = jnp.clip(y0i + 1, 0, Hh - 1)
    z0i = jnp.clip(z0.astype(jnp.int32), 0, Dd - 1); z1i = jnp.clip(z0i + 1, 0, Dd - 1)

    def g(zi, yi, xi):
        return feat[:, zi, yi, xi]

    c00 = g(z0i, y0i, x0i) * (1 - xd) + g(z0i, y0i, x1i) * xd
    c01 = g(z1i, y0i, x0i) * (1 - xd) + g(z1i, y0i, x1i) * xd
    c10 = g(z0i, y1i, x0i) * (1 - xd) + g(z0i, y1i, x1i) * xd
    c11 = g(z1i, y1i, x0i) * (1 - xd) + g(z1i, y1i, x1i) * xd
    c0 = c00 * (1 - yd) + c10 * yd
    c1 = c01 * (1 - yd) + c11 * yd
    return (c0 * (1 - zd) + c1 * zd).T


def kernel(t, Vt, edge_index, in_graph_features, skip0, skip1, params):
    batch_size, Vn, Dn = Vt.shape
    sampled0 = jax.vmap(_trilinear)(skip0, Vt)
    sampled1 = jax.vmap(_trilinear)(skip1, Vt)
    latent = jnp.concatenate([Vt, in_graph_features, sampled0, sampled1], axis=2)
    x = latent.reshape(batch_size * Vn, -1)
    src = edge_index[0]
    dst = edge_index[1]
    n = x.shape[0]

    def neigh_sum(h):
        return jax.ops.segment_sum(jnp.take(h, src, axis=0), dst, num_segments=n)

    for block in params['blocks']:
        h = x
        for (W0, W1, b) in block['layers']:
            agg = neigh_sum(h)
            h = _dense_layer(h, agg, W0, W1, b, relu=True)
        res = x @ block['proj'] if block['proj'] is not None else x
        x = h + res
    W0, W1, b = params['f2v']
    agg = neigh_sum(x)
    dV = _dense_layer(x, agg, W0, W1, b, relu=False)
    return dV.reshape(batch_size, Vn, Dn)


# speedup vs baseline: 8.2751x; 8.2299x over previous
import functools

import jax
import jax.numpy as jnp
from jax import lax
from jax.experimental import pallas as pl
from jax.experimental.pallas import tpu as pltpu
from jax.experimental.pallas import tpu_sc as plsc

_N = 100000
_E = 1600000
_NSUB = 16
_NIDX = 8
_SUPER = _NIDX * 128
_NSUPER = 98
_EPS = _SUPER * _NSUPER
_EPAD = _EPS * _NSUB
_PADROWS = 128
_NOUT = 100096
_YROWS = _NOUT + _PADROWS
_ZCH = _NOUT // _NSUB
_CH = 16


def _spmm_body(x0, x1, srcr, dstr, zroz, y0, y1,
               src_i, dst_i, rows_v, y_sh, gsem, ssem):
    c = lax.axis_index("c")
    s = lax.axis_index("s")
    pltpu.sync_copy(zroz, y_sh.at[pl.ds(s * _ZCH, _ZCH)])
    plsc.subcore_barrier()

    def body(i, carry):
        row0 = (s * _NSUPER + i) * _NIDX
        pltpu.sync_copy(srcr.at[pl.ds(row0, _NIDX)], src_i)
        pltpu.sync_copy(dstr.at[pl.ds(row0, _NIDX)], dst_i)

        @pl.when(c == 0)
        def _():
            d = [pltpu.async_copy(x0.at[src_i.at[j]],
                                  rows_v.at[pl.ds(j * 128, 128)], gsem)
                 for j in range(_NIDX)]
            for dd in d:
                dd.wait()

        @pl.when(c == 1)
        def _():
            d = [pltpu.async_copy(x1.at[src_i.at[j]],
                                  rows_v.at[pl.ds(j * 128, 128)], gsem)
                 for j in range(_NIDX)]
            for dd in d:
                dd.wait()

        d = [pltpu.async_copy(rows_v.at[pl.ds(j * 128, 128)],
                              y_sh.at[dst_i.at[j]], ssem, add=True)
             for j in range(_NIDX)]
        for dd in d:
            dd.wait()
        return carry

    lax.fori_loop(0, _NSUPER, body, 0)
    plsc.subcore_barrier()

    @pl.when(c == 0)
    def _():
        pltpu.sync_copy(y_sh.at[pl.ds(s * _ZCH, _ZCH)],
                        y0.at[pl.ds(s * _ZCH, _ZCH)])

    @pl.when(c == 1)
    def _():
        pltpu.sync_copy(y_sh.at[pl.ds(s * _ZCH, _ZCH)],
                        y1.at[pl.ds(s * _ZCH, _ZCH)])


_sc_mesh = plsc.VectorSubcoreMesh(core_axis_name="c", subcore_axis_name="s",
                                  num_cores=2, num_subcores=_NSUB)

_spmm_call = pl.kernel(
    _spmm_body,
    mesh=_sc_mesh,
    out_type=(jax.ShapeDtypeStruct((_NOUT, _CH), jnp.float32),
              jax.ShapeDtypeStruct((_NOUT, _CH), jnp.float32)),
    scratch_types=[
        pltpu.VMEM((_NIDX, 128), jnp.int32),
        pltpu.VMEM((_NIDX, 128), jnp.int32),
        pltpu.VMEM((_SUPER, _CH), jnp.float32),
        pltpu.VMEM_SHARED((_YROWS, _CH), jnp.float32),
        pltpu.SemaphoreType.DMA,
        pltpu.SemaphoreType.DMA,
    ],
    compiler_params=pltpu.CompilerParams(use_tc_tiling_on_sc=False),
)


def _spmm32(p, src2d, dst2d, zro):
    y0, y1 = _spmm_call(p[:, :_CH], p[:, _CH:], src2d, dst2d, zro)
    return jnp.concatenate([y0[:_N], y1[:_N]], axis=1)


def _mm_body(x_ref, w_ref, o_ref):
    o_ref[...] = x_ref[...] @ w_ref[...]


def _mm(x, W):
    n, d = x.shape
    c_out = W.shape[1]
    blk = 2000
    return pl.pallas_call(
        _mm_body,
        grid=(n // blk,),
        in_specs=[
            pl.BlockSpec((blk, d), lambda i: (i, 0)),
            pl.BlockSpec((d, c_out), lambda i: (0, 0)),
        ],
        out_specs=pl.BlockSpec((blk, c_out), lambda i: (i, 0)),
        out_shape=jax.ShapeDtypeStruct((n, c_out), jnp.float32),
    )(x, W)


def _combine_body(x_ref, agg_ref, w0_ref, b_ref, o_ref, *, relu):
    acc = x_ref[...] @ w0_ref[...] + agg_ref[...] + b_ref[...]
    if relu:
        acc = jnp.maximum(acc, 0.0)
    o_ref[...] = acc


def _combine(x, agg, W0, b, relu):
    n, d = x.shape
    c_out = W0.shape[1]
    blk = 2000
    return pl.pallas_call(
        functools.partial(_combine_body, relu=relu),
        grid=(n // blk,),
        in_specs=[
            pl.BlockSpec((blk, d), lambda i: (i, 0)),
            pl.BlockSpec((blk, c_out), lambda i: (i, 0)),
            pl.BlockSpec((d, c_out), lambda i: (0, 0)),
            pl.BlockSpec((1, c_out), lambda i: (0, 0)),
        ],
        out_specs=pl.BlockSpec((blk, c_out), lambda i: (i, 0)),
        out_shape=jax.ShapeDtypeStruct((n, c_out), jnp.float32),
    )(x, agg, W0, b.reshape(1, -1))


def _trilinear(feat, coords):
    C, Dd, Hh, Ww = feat.shape
    x = (coords[:, 0] + 1.0) * 0.5 * (Ww - 1)
    y = (coords[:, 1] + 1.0) * 0.5 * (Hh - 1)
    z = (coords[:, 2] + 1.0) * 0.5 * (Dd - 1)
    x0 = jnp.floor(x); y0 = jnp.floor(y); z0 = jnp.floor(z)
    xd = (x - x0)[None, :]; yd = (y - y0)[None, :]; zd = (z - z0)[None, :]
    x0i = jnp.clip(x0.astype(jnp.int32), 0, Ww - 1); x1i = jnp.clip(x0i + 1, 0, Ww - 1)
    y0i = jnp.clip(y0.astype(jnp.int32), 0, Hh - 1); y1i = jnp.clip(y0i + 1, 0, Hh - 1)
    z0i = jnp.clip(z0.astype(jnp.int32), 0, Dd - 1); z1i = jnp.clip(z0i + 1, 0, Dd - 1)

    def g(zi, yi, xi):
        return feat[:, zi, yi, xi]

    c00 = g(z0i, y0i, x0i) * (1 - xd) + g(z0i, y0i, x1i) * xd
    c01 = g(z1i, y0i, x0i) * (1 - xd) + g(z1i, y0i, x1i) * xd
    c10 = g(z0i, y1i, x0i) * (1 - xd) + g(z0i, y1i, x1i) * xd
    c11 = g(z1i, y1i, x0i) * (1 - xd) + g(z1i, y1i, x1i) * xd
    c0 = c00 * (1 - yd) + c10 * yd
    c1 = c01 * (1 - yd) + c11 * yd
    return (c0 * (1 - zd) + c1 * zd).T


def kernel(t, Vt, edge_index, in_graph_features, skip0, skip1, params):
    batch_size, Vn, Dn = Vt.shape
    sampled0 = jax.vmap(_trilinear)(skip0, Vt)
    sampled1 = jax.vmap(_trilinear)(skip1, Vt)
    latent = jnp.concatenate([Vt, in_graph_features, sampled0, sampled1], axis=2)
    x = latent.reshape(batch_size * Vn, -1)

    src = edge_index[0]
    dst = edge_index[1]
    pad = _EPAD - _E
    ar = jnp.arange(pad, dtype=jnp.int32)
    src2d = jnp.concatenate([src, ar % _N]).reshape(-1, 128)
    dst2d = jnp.concatenate([dst, _NOUT + (ar % _PADROWS)]).reshape(-1, 128)
    zro = jnp.zeros((_ZCH, _CH), jnp.float32)

    for block in params['blocks']:
        h = x
        for (W0, W1, b) in block['layers']:
            p = _mm(h, W1)
            agg = _spmm32(p, src2d, dst2d, zro)
            h = _combine(h, agg, W0, b, relu=True)
        res = _mm(x, block['proj']) if block['proj'] is not None else x
        x = h + res
    W0, W1, b = params['f2v']
    p = jnp.pad(_mm(x, W1), ((0, 0), (0, 32 - Dn)))
    agg = _spmm32(p, src2d, dst2d, zro)[:, :Dn]
    dV = _combine(x, agg, W0, b, relu=False)
    return dV.reshape(batch_size, Vn, Dn)


# speedup vs baseline: 8.2787x; 1.0004x over previous
import functools

import jax
import jax.numpy as jnp
from jax import lax
from jax.experimental import pallas as pl
from jax.experimental.pallas import tpu as pltpu
from jax.experimental.pallas import tpu_sc as plsc

_N = 100000
_E = 1600000
_NSUB = 16
_SUPER = 1024
_NSUPER = 98
_EPS = _SUPER * _NSUPER
_EPAD = _EPS * _NSUB
_PADROWS = 128
_NOUT = 100096
_YROWS = _NOUT + _PADROWS
_ZCH = _NOUT // _NSUB
_CH = 16


def _spmm_body(x0, x1, srcr, dstr, zroz, y0, y1,
               src_i, dst_i, rows_v, y_sh, gsem, ssem):
    c = lax.axis_index("c")
    s = lax.axis_index("s")
    pltpu.sync_copy(zroz, y_sh.at[pl.ds(s * _ZCH, _ZCH)])
    plsc.subcore_barrier()

    def body(i, carry):
        e0 = (s * _NSUPER + i) * _SUPER
        pltpu.sync_copy(srcr.at[pl.ds(e0, _SUPER)], src_i)
        pltpu.sync_copy(dstr.at[pl.ds(e0, _SUPER)], dst_i)

        @pl.when(c == 0)
        def _():
            pltpu.async_copy(x0.at[src_i], rows_v, gsem).wait()

        @pl.when(c == 1)
        def _():
            pltpu.async_copy(x1.at[src_i], rows_v, gsem).wait()

        pltpu.async_copy(rows_v, y_sh.at[dst_i], ssem, add=True).wait()
        return carry

    lax.fori_loop(0, _NSUPER, body, 0)
    plsc.subcore_barrier()

    @pl.when(c == 0)
    def _():
        pltpu.sync_copy(y_sh.at[pl.ds(s * _ZCH, _ZCH)],
                        y0.at[pl.ds(s * _ZCH, _ZCH)])

    @pl.when(c == 1)
    def _():
        pltpu.sync_copy(y_sh.at[pl.ds(s * _ZCH, _ZCH)],
                        y1.at[pl.ds(s * _ZCH, _ZCH)])


_sc_mesh = plsc.VectorSubcoreMesh(core_axis_name="c", subcore_axis_name="s",
                                  num_cores=2, num_subcores=_NSUB)

_spmm_call = pl.kernel(
    _spmm_body,
    mesh=_sc_mesh,
    out_type=(jax.ShapeDtypeStruct((_NOUT, _CH), jnp.float32),
              jax.ShapeDtypeStruct((_NOUT, _CH), jnp.float32)),
    scratch_types=[
        pltpu.VMEM((_SUPER,), jnp.int32),
        pltpu.VMEM((_SUPER,), jnp.int32),
        pltpu.VMEM((_SUPER, _CH), jnp.float32),
        pltpu.VMEM_SHARED((_YROWS, _CH), jnp.float32),
        pltpu.SemaphoreType.DMA,
        pltpu.SemaphoreType.DMA,
    ],
    compiler_params=pltpu.CompilerParams(use_tc_tiling_on_sc=False),
)


def _spmm32(p, src2d, dst2d, zro):
    y0, y1 = _spmm_call(p[:, :_CH], p[:, _CH:], src2d, dst2d, zro)
    return jnp.concatenate([y0[:_N], y1[:_N]], axis=1)


def _mm_body(x_ref, w_ref, o_ref):
    o_ref[...] = x_ref[...] @ w_ref[...]


def _mm(x, W):
    n, d = x.shape
    c_out = W.shape[1]
    blk = 2000
    return pl.pallas_call(
        _mm_body,
        grid=(n // blk,),
        in_specs=[
            pl.BlockSpec((blk, d), lambda i: (i, 0)),
            pl.BlockSpec((d, c_out), lambda i: (0, 0)),
        ],
        out_specs=pl.BlockSpec((blk, c_out), lambda i: (i, 0)),
        out_shape=jax.ShapeDtypeStruct((n, c_out), jnp.float32),
    )(x, W)


def _combine_body(x_ref, agg_ref, w0_ref, b_ref, o_ref, *, relu):
    acc = x_ref[...] @ w0_ref[...] + agg_ref[...] + b_ref[...]
    if relu:
        acc = jnp.maximum(acc, 0.0)
    o_ref[...] = acc


def _combine(x, agg, W0, b, relu):
    n, d = x.shape
    c_out = W0.shape[1]
    blk = 2000
    return pl.pallas_call(
        functools.partial(_combine_body, relu=relu),
        grid=(n // blk,),
        in_specs=[
            pl.BlockSpec((blk, d), lambda i: (i, 0)),
            pl.BlockSpec((blk, c_out), lambda i: (i, 0)),
            pl.BlockSpec((d, c_out), lambda i: (0, 0)),
            pl.BlockSpec((1, c_out), lambda i: (0, 0)),
        ],
        out_specs=pl.BlockSpec((blk, c_out), lambda i: (i, 0)),
        out_shape=jax.ShapeDtypeStruct((n, c_out), jnp.float32),
    )(x, agg, W0, b.reshape(1, -1))


def _trilinear(feat, coords):
    C, Dd, Hh, Ww = feat.shape
    x = (coords[:, 0] + 1.0) * 0.5 * (Ww - 1)
    y = (coords[:, 1] + 1.0) * 0.5 * (Hh - 1)
    z = (coords[:, 2] + 1.0) * 0.5 * (Dd - 1)
    x0 = jnp.floor(x); y0 = jnp.floor(y); z0 = jnp.floor(z)
    xd = (x - x0)[None, :]; yd = (y - y0)[None, :]; zd = (z - z0)[None, :]
    x0i = jnp.clip(x0.astype(jnp.int32), 0, Ww - 1); x1i = jnp.clip(x0i + 1, 0, Ww - 1)
    y0i = jnp.clip(y0.astype(jnp.int32), 0, Hh - 1); y1i = jnp.clip(y0i + 1, 0, Hh - 1)
    z0i = jnp.clip(z0.astype(jnp.int32), 0, Dd - 1); z1i = jnp.clip(z0i + 1, 0, Dd - 1)

    def g(zi, yi, xi):
        return feat[:, zi, yi, xi]

    c00 = g(z0i, y0i, x0i) * (1 - xd) + g(z0i, y0i, x1i) * xd
    c01 = g(z1i, y0i, x0i) * (1 - xd) + g(z1i, y0i, x1i) * xd
    c10 = g(z0i, y1i, x0i) * (1 - xd) + g(z0i, y1i, x1i) * xd
    c11 = g(z1i, y1i, x0i) * (1 - xd) + g(z1i, y1i, x1i) * xd
    c0 = c00 * (1 - yd) + c10 * yd
    c1 = c01 * (1 - yd) + c11 * yd
    return (c0 * (1 - zd) + c1 * zd).T


def kernel(t, Vt, edge_index, in_graph_features, skip0, skip1, params):
    batch_size, Vn, Dn = Vt.shape
    sampled0 = jax.vmap(_trilinear)(skip0, Vt)
    sampled1 = jax.vmap(_trilinear)(skip1, Vt)
    latent = jnp.concatenate([Vt, in_graph_features, sampled0, sampled1], axis=2)
    x = latent.reshape(batch_size * Vn, -1)

    src = edge_index[0]
    dst = edge_index[1]
    pad = _EPAD - _E
    ar = jnp.arange(pad, dtype=jnp.int32)
    src2d = jnp.concatenate([src, ar % _N])
    dst2d = jnp.concatenate([dst, _NOUT + (ar % _PADROWS)])
    zro = jnp.zeros((_ZCH, _CH), jnp.float32)

    for block in params['blocks']:
        h = x
        for (W0, W1, b) in block['layers']:
            p = _mm(h, W1)
            agg = _spmm32(p, src2d, dst2d, zro)
            h = _combine(h, agg, W0, b, relu=True)
        res = _mm(x, block['proj']) if block['proj'] is not None else x
        x = h + res
    W0, W1, b = params['f2v']
    p = jnp.pad(_mm(x, W1), ((0, 0), (0, 32 - Dn)))
    agg = _spmm32(p, src2d, dst2d, zro)[:, :Dn]
    dV = _combine(x, agg, W0, b, relu=False)
    return dV.reshape(batch_size, Vn, Dn)


# speedup vs baseline: 8.5440x; 1.0320x over previous
import functools

import jax
import jax.numpy as jnp
from jax import lax
from jax.experimental import pallas as pl
from jax.experimental.pallas import tpu as pltpu
from jax.experimental.pallas import tpu_sc as plsc

_N = 100000
_E = 1600000
_NSUB = 16
_SUPER = 1024
_NSUPER = 98
_EPS = _SUPER * _NSUPER
_EPAD = _EPS * _NSUB
_PADROWS = 128
_NOUT = 100096
_YROWS = _NOUT + _PADROWS
_ZCH = _NOUT // _NSUB
_CH = 16


def _spmm_body(x0, x1, srcr, dstr, zroz, y0, y1,
               src_i, dst_i, rows_v, y_sh, gsem, ssem):
    c = lax.axis_index("c")
    s = lax.axis_index("s")
    pltpu.sync_copy(zroz, y_sh.at[pl.ds(s * _ZCH, _ZCH)])
    plsc.subcore_barrier()

    def body(i, carry):
        e0 = (s * _NSUPER + i) * _SUPER
        pltpu.sync_copy(srcr.at[pl.ds(e0, _SUPER)], src_i)
        pltpu.sync_copy(dstr.at[pl.ds(e0, _SUPER)], dst_i)

        @pl.when(c == 0)
        def _():
            pltpu.async_copy(x0.at[src_i], rows_v, gsem).wait()

        @pl.when(c == 1)
        def _():
            pltpu.async_copy(x1.at[src_i], rows_v, gsem).wait()

        pltpu.async_copy(rows_v, y_sh.at[dst_i], ssem, add=True).wait()
        return carry

    lax.fori_loop(0, _NSUPER, body, 0)
    plsc.subcore_barrier()

    @pl.when(c == 0)
    def _():
        pltpu.sync_copy(y_sh.at[pl.ds(s * _ZCH, _ZCH)],
                        y0.at[pl.ds(s * _ZCH, _ZCH)])

    @pl.when(c == 1)
    def _():
        pltpu.sync_copy(y_sh.at[pl.ds(s * _ZCH, _ZCH)],
                        y1.at[pl.ds(s * _ZCH, _ZCH)])


_sc_mesh = plsc.VectorSubcoreMesh(core_axis_name="c", subcore_axis_name="s",
                                  num_cores=2, num_subcores=_NSUB)

_spmm_call = pl.kernel(
    _spmm_body,
    mesh=_sc_mesh,
    out_type=(jax.ShapeDtypeStruct((_NOUT, _CH), jnp.float32),
              jax.ShapeDtypeStruct((_NOUT, _CH), jnp.float32)),
    scratch_types=[
        pltpu.VMEM((_SUPER,), jnp.int32),
        pltpu.VMEM((_SUPER,), jnp.int32),
        pltpu.VMEM((_SUPER, _CH), jnp.float32),
        pltpu.VMEM_SHARED((_YROWS, _CH), jnp.float32),
        pltpu.SemaphoreType.DMA,
        pltpu.SemaphoreType.DMA,
    ],
    compiler_params=pltpu.CompilerParams(use_tc_tiling_on_sc=False),
)


def _spmm32(p, src2d, dst2d, zro):
    y0, y1 = _spmm_call(p[:, :_CH], p[:, _CH:], src2d, dst2d, zro)
    return jnp.concatenate([y0[:_N], y1[:_N]], axis=1)


_VP = 102400
_VW = 3200
_VC = 128
_NVC = _VW // _VC


def _tri_body(t0, t1, idx0, w0, idx1, w1, o0, o1,
              idx0b, w0b, idx1b, w1b, buf0, buf1, ob0, ob1, gsem):
    c = lax.axis_index("c")
    s = lax.axis_index("s")
    wkr = s * 2 + c
    lane = lax.iota(jnp.int32, 16)
    rbase = lane >> 3
    cbase = lane & 7

    def it(i, carry):
        col0 = wkr * _VW + i * _VC
        pltpu.sync_copy(idx0.at[:, pl.ds(col0, _VC)], idx0b)
        pltpu.sync_copy(w0.at[:, pl.ds(col0, _VC)], w0b)
        pltpu.sync_copy(idx1.at[:, pl.ds(col0, _VC)], idx1b)
        pltpu.sync_copy(w1.at[:, pl.ds(col0, _VC)], w1b)
        d = [pltpu.async_copy(t0.at[idx0b.at[k]], buf0.at[k], gsem)
             for k in range(8)]
        d += [pltpu.async_copy(t1.at[idx1b.at[k]], buf1.at[k], gsem)
              for k in range(8)]
        for dd in d:
            dd.wait()

        def vbody(v2, carry2):
            va = 2 * v2
            vb = va + 1
            for v in (va, vb):
                vs = jnp.full((16,), v, jnp.int32)
                acc = jnp.zeros((16,), jnp.float32)
                for k in range(8):
                    ks = jnp.full((16,), k, jnp.int32)
                    wv = plsc.load_gather(w1b, [ks, vs])
                    acc = acc + wv * buf1[k, v, :]
                ob1[v, :] = acc
            ri = va + rbase
            acc0 = jnp.zeros((16,), jnp.float32)
            for k in range(8):
                ks = jnp.full((16,), k, jnp.int32)
                wv = plsc.load_gather(w0b, [ks, ri])
                val = plsc.load_gather(buf0, [ks, ri, cbase])
                acc0 = acc0 + wv * val
            plsc.store_scatter(ob0, [ri, cbase], acc0)
            return carry2

        lax.fori_loop(0, _VC // 2, vbody, 0)
        pltpu.sync_copy(ob0, o0.at[pl.ds(col0, _VC)])
        pltpu.sync_copy(ob1, o1.at[pl.ds(col0, _VC)])
        return carry

    lax.fori_loop(0, _NVC, it, 0)


_tri_call = pl.kernel(
    _tri_body,
    mesh=_sc_mesh,
    out_type=(jax.ShapeDtypeStruct((_VP, 8), jnp.float32),
              jax.ShapeDtypeStruct((_VP, 16), jnp.float32)),
    scratch_types=[
        pltpu.VMEM((8, _VC), jnp.int32),
        pltpu.VMEM((8, _VC), jnp.float32),
        pltpu.VMEM((8, _VC), jnp.int32),
        pltpu.VMEM((8, _VC), jnp.float32),
        pltpu.VMEM((8, _VC, 8), jnp.float32),
        pltpu.VMEM((8, _VC, 16), jnp.float32),
        pltpu.VMEM((_VC, 8), jnp.float32),
        pltpu.VMEM((_VC, 16), jnp.float32),
        pltpu.SemaphoreType.DMA,
    ],
    compiler_params=pltpu.CompilerParams(use_tc_tiling_on_sc=False,
                                         needs_layout_passes=False),
)


def _tri_prep(coords, res):
    x = (coords[:, 0] + 1.0) * 0.5 * (res - 1)
    y = (coords[:, 1] + 1.0) * 0.5 * (res - 1)
    z = (coords[:, 2] + 1.0) * 0.5 * (res - 1)
    x0 = jnp.floor(x); y0 = jnp.floor(y); z0 = jnp.floor(z)
    xd = x - x0; yd = y - y0; zd = z - z0
    x0i = jnp.clip(x0.astype(jnp.int32), 0, res - 1); x1i = jnp.clip(x0i + 1, 0, res - 1)
    y0i = jnp.clip(y0.astype(jnp.int32), 0, res - 1); y1i = jnp.clip(y0i + 1, 0, res - 1)
    z0i = jnp.clip(z0.astype(jnp.int32), 0, res - 1); z1i = jnp.clip(z0i + 1, 0, res - 1)
    idxs, ws = [], []
    for dz in (0, 1):
        zi = z1i if dz else z0i
        wz = zd if dz else 1.0 - zd
        for dy in (0, 1):
            yi = y1i if dy else y0i
            wy = yd if dy else 1.0 - yd
            for dx in (0, 1):
                xi = x1i if dx else x0i
                wx = xd if dx else 1.0 - xd
                idxs.append((zi * res + yi) * res + xi)
                ws.append(wz * wy * wx)
    return jnp.stack(idxs), jnp.stack(ws)


def _sample_skips(Vt, skip0, skip1):
    b, v, _ = Vt.shape
    n = b * v
    coords = Vt.reshape(n, 3)
    boff = (jnp.arange(n, dtype=jnp.int32) // v)
    pad = _VP - n
    pr = jnp.arange(pad, dtype=jnp.int32)

    def level(skip, res):
        ch = skip.shape[1]
        tab = skip.transpose(0, 2, 3, 4, 1).reshape(-1, ch)
        idx, w = _tri_prep(coords, res)
        idx = idx + (boff * (res * res * res))[None, :]
        idx = jnp.concatenate([idx, jnp.broadcast_to(pr % (res * res), (8, pad))], axis=1)
        w = jnp.concatenate([w, jnp.zeros((8, pad), jnp.float32)], axis=1)
        return tab, idx, w

    t0, idx0, w0 = level(skip0, skip0.shape[2])
    t1, idx1, w1 = level(skip1, skip1.shape[2])
    o0, o1 = _tri_call(t0, t1, idx0, w0, idx1, w1)
    return (o0[:n].reshape(b, v, 8), o1[:n].reshape(b, v, 16))


def _mm_body(x_ref, w_ref, o_ref):
    o_ref[...] = x_ref[...] @ w_ref[...]


def _mm(x, W):
    n, d = x.shape
    c_out = W.shape[1]
    blk = 2000
    return pl.pallas_call(
        _mm_body,
        grid=(n // blk,),
        in_specs=[
            pl.BlockSpec((blk, d), lambda i: (i, 0)),
            pl.BlockSpec((d, c_out), lambda i: (0, 0)),
        ],
        out_specs=pl.BlockSpec((blk, c_out), lambda i: (i, 0)),
        out_shape=jax.ShapeDtypeStruct((n, c_out), jnp.float32),
    )(x, W)


def _combine_body(x_ref, agg_ref, w0_ref, b_ref, o_ref, *, relu):
    acc = x_ref[...] @ w0_ref[...] + agg_ref[...] + b_ref[...]
    if relu:
        acc = jnp.maximum(acc, 0.0)
    o_ref[...] = acc


def _combine(x, agg, W0, b, relu):
    n, d = x.shape
    c_out = W0.shape[1]
    blk = 2000
    return pl.pallas_call(
        functools.partial(_combine_body, relu=relu),
        grid=(n // blk,),
        in_specs=[
            pl.BlockSpec((blk, d), lambda i: (i, 0)),
            pl.BlockSpec((blk, c_out), lambda i: (i, 0)),
            pl.BlockSpec((d, c_out), lambda i: (0, 0)),
            pl.BlockSpec((1, c_out), lambda i: (0, 0)),
        ],
        out_specs=pl.BlockSpec((blk, c_out), lambda i: (i, 0)),
        out_shape=jax.ShapeDtypeStruct((n, c_out), jnp.float32),
    )(x, agg, W0, b.reshape(1, -1))


def kernel(t, Vt, edge_index, in_graph_features, skip0, skip1, params):
    batch_size, Vn, Dn = Vt.shape
    sampled0, sampled1 = _sample_skips(Vt, skip0, skip1)
    latent = jnp.concatenate([Vt, in_graph_features, sampled0, sampled1], axis=2)
    x = latent.reshape(batch_size * Vn, -1)

    src = edge_index[0]
    dst = edge_index[1]
    pad = _EPAD - _E
    ar = jnp.arange(pad, dtype=jnp.int32)
    src2d = jnp.concatenate([src, ar % _N])
    dst2d = jnp.concatenate([dst, _NOUT + (ar % _PADROWS)])
    zro = jnp.zeros((_ZCH, _CH), jnp.float32)

    for block in params['blocks']:
        h = x
        for (W0, W1, b) in block['layers']:
            p = _mm(h, W1)
            agg = _spmm32(p, src2d, dst2d, zro)
            h = _combine(h, agg, W0, b, relu=True)
        res = _mm(x, block['proj']) if block['proj'] is not None else x
        x = h + res
    W0, W1, b = params['f2v']
    p = jnp.pad(_mm(x, W1), ((0, 0), (0, 32 - Dn)))
    agg = _spmm32(p, src2d, dst2d, zro)[:, :Dn]
    dV = _combine(x, agg, W0, b, relu=False)
    return dV.reshape(batch_size, Vn, Dn)


# speedup vs baseline: 10.9766x; 1.2847x over previous
import functools

import jax
import jax.numpy as jnp
from jax import lax
from jax.experimental import pallas as pl
from jax.experimental.pallas import tpu as pltpu
from jax.experimental.pallas import tpu_sc as plsc

_N = 100000
_E = 1600000
_R = 102400
_NSUB = 16
_SUPER = 1024
_NSUPER = 98
_EPS = _SUPER * _NSUPER
_EPAD = _EPS * _NSUB
_PADROWS = 128
_YROWS = _R + _PADROWS
_ZCH = _R // _NSUB
_CH = 16

_sc_mesh = plsc.VectorSubcoreMesh(core_axis_name="c", subcore_axis_name="s",
                                  num_cores=2, num_subcores=_NSUB)


def _spmm_body(x0, x1, srcr, dstr, zroz, y0, y1,
               src_i, dst_i, rows_v, y_sh, gsem, ssem):
    c = lax.axis_index("c")
    s = lax.axis_index("s")
    pltpu.sync_copy(zroz, y_sh.at[pl.ds(s * _ZCH, _ZCH)])
    plsc.subcore_barrier()

    def body(i, carry):
        e0 = (s * _NSUPER + i) * _SUPER
        pltpu.sync_copy(srcr.at[pl.ds(e0, _SUPER)], src_i)
        pltpu.sync_copy(dstr.at[pl.ds(e0, _SUPER)], dst_i)

        @pl.when(c == 0)
        def _():
            pltpu.async_copy(x0.at[src_i], rows_v, gsem).wait()

        @pl.when(c == 1)
        def _():
            pltpu.async_copy(x1.at[src_i], rows_v, gsem).wait()

        pltpu.async_copy(rows_v, y_sh.at[dst_i], ssem, add=True).wait()
        return carry

    lax.fori_loop(0, _NSUPER, body, 0)
    plsc.subcore_barrier()

    @pl.when(c == 0)
    def _():
        pltpu.sync_copy(y_sh.at[pl.ds(s * _ZCH, _ZCH)],
                        y0.at[pl.ds(s * _ZCH, _ZCH)])

    @pl.when(c == 1)
    def _():
        pltpu.sync_copy(y_sh.at[pl.ds(s * _ZCH, _ZCH)],
                        y1.at[pl.ds(s * _ZCH, _ZCH)])


_spmm_call = pl.kernel(
    _spmm_body,
    mesh=_sc_mesh,
    out_type=(jax.ShapeDtypeStruct((_R, _CH), jnp.float32),
              jax.ShapeDtypeStruct((_R, _CH), jnp.float32)),
    scratch_types=[
        pltpu.VMEM((_SUPER,), jnp.int32),
        pltpu.VMEM((_SUPER,), jnp.int32),
        pltpu.VMEM((_SUPER, _CH), jnp.float32),
        pltpu.VMEM_SHARED((_YROWS, _CH), jnp.float32),
        pltpu.SemaphoreType.DMA,
        pltpu.SemaphoreType.DMA,
    ],
    compiler_params=pltpu.CompilerParams(use_tc_tiling_on_sc=False),
)


_VW = _R // 32
_VC = 128
_NVC = _VW // _VC


def _tri_body(t0, t1, idx0, w0, idx1, w1, o0, o1,
              idx0b, w0b, idx1b, w1b, buf0, buf1, ob0, ob1, gsem):
    c = lax.axis_index("c")
    s = lax.axis_index("s")
    wkr = s * 2 + c
    lane = lax.iota(jnp.int32, 16)
    rbase = lane >> 3
    cbase = lane & 7

    def it(i, carry):
        col0 = wkr * _VW + i * _VC
        pltpu.sync_copy(idx0.at[:, pl.ds(col0, _VC)], idx0b)
        pltpu.sync_copy(w0.at[:, pl.ds(col0, _VC)], w0b)
        pltpu.sync_copy(idx1.at[:, pl.ds(col0, _VC)], idx1b)
        pltpu.sync_copy(w1.at[:, pl.ds(col0, _VC)], w1b)
        d = [pltpu.async_copy(t0.at[idx0b.at[k]], buf0.at[k], gsem)
             for k in range(8)]
        d += [pltpu.async_copy(t1.at[idx1b.at[k]], buf1.at[k], gsem)
              for k in range(8)]
        for dd in d:
            dd.wait()

        def vbody(v2, carry2):
            va = 2 * v2
            vb = va + 1
            for v in (va, vb):
                vs = jnp.full((16,), v, jnp.int32)
                acc = jnp.zeros((16,), jnp.float32)
                for k in range(8):
                    ks = jnp.full((16,), k, jnp.int32)
                    wv = plsc.load_gather(w1b, [ks, vs])
                    acc = acc + wv * buf1[k, v, :]
                ob1[v, :] = acc
            ri = va + rbase
            acc0 = jnp.zeros((16,), jnp.float32)
            for k in range(8):
                ks = jnp.full((16,), k, jnp.int32)
                wv = plsc.load_gather(w0b, [ks, ri])
                val = plsc.load_gather(buf0, [ks, ri, cbase])
                acc0 = acc0 + wv * val
            plsc.store_scatter(ob0, [ri, cbase], acc0)
            return carry2

        lax.fori_loop(0, _VC // 2, vbody, 0)
        pltpu.sync_copy(ob0, o0.at[pl.ds(col0, _VC)])
        pltpu.sync_copy(ob1, o1.at[pl.ds(col0, _VC)])
        return carry

    lax.fori_loop(0, _NVC, it, 0)


_tri_call = pl.kernel(
    _tri_body,
    mesh=_sc_mesh,
    out_type=(jax.ShapeDtypeStruct((_R, 8), jnp.float32),
              jax.ShapeDtypeStruct((_R, 16), jnp.float32)),
    scratch_types=[
        pltpu.VMEM((8, _VC), jnp.int32),
        pltpu.VMEM((8, _VC), jnp.float32),
        pltpu.VMEM((8, _VC), jnp.int32),
        pltpu.VMEM((8, _VC), jnp.float32),
        pltpu.VMEM((8, _VC, 8), jnp.float32),
        pltpu.VMEM((8, _VC, 16), jnp.float32),
        pltpu.VMEM((_VC, 8), jnp.float32),
        pltpu.VMEM((_VC, 16), jnp.float32),
        pltpu.SemaphoreType.DMA,
    ],
    compiler_params=pltpu.CompilerParams(use_tc_tiling_on_sc=False,
                                         needs_layout_passes=False),
)


def _tri_prep(coords, res):
    x = (coords[:, 0] + 1.0) * 0.5 * (res - 1)
    y = (coords[:, 1] + 1.0) * 0.5 * (res - 1)
    z = (coords[:, 2] + 1.0) * 0.5 * (res - 1)
    x0 = jnp.floor(x); y0 = jnp.floor(y); z0 = jnp.floor(z)
    xd = x - x0; yd = y - y0; zd = z - z0
    x0i = jnp.clip(x0.astype(jnp.int32), 0, res - 1); x1i = jnp.clip(x0i + 1, 0, res - 1)
    y0i = jnp.clip(y0.astype(jnp.int32), 0, res - 1); y1i = jnp.clip(y0i + 1, 0, res - 1)
    z0i = jnp.clip(z0.astype(jnp.int32), 0, res - 1); z1i = jnp.clip(z0i + 1, 0, res - 1)
    idxs, ws = [], []
    for dz in (0, 1):
        zi = z1i if dz else z0i
        wz = zd if dz else 1.0 - zd
        for dy in (0, 1):
            yi = y1i if dy else y0i
            wy = yd if dy else 1.0 - yd
            for dx in (0, 1):
                xi = x1i if dx else x0i
                wx = xd if dx else 1.0 - xd
                idxs.append((zi * res + yi) * res + xi)
                ws.append(wz * wy * wx)
    return jnp.stack(idxs), jnp.stack(ws)


def _sample_skips(Vt, skip0, skip1):
    b, v, _ = Vt.shape
    n = b * v
    coords = Vt.reshape(n, 3)
    boff = (jnp.arange(n, dtype=jnp.int32) // v)
    pad = _R - n
    pr = jnp.arange(pad, dtype=jnp.int32)

    def level(skip, res):
        ch = skip.shape[1]
        tab = skip.transpose(0, 2, 3, 4, 1).reshape(-1, ch)
        idx, w = _tri_prep(coords, res)
        idx = idx + (boff * (res * res * res))[None, :]
        idx = jnp.concatenate([idx, jnp.broadcast_to(pr % (res * res), (8, pad))], axis=1)
        w = jnp.concatenate([w, jnp.zeros((8, pad), jnp.float32)], axis=1)
        return tab, idx, w

    t0, idx0, w0 = level(skip0, skip0.shape[2])
    t1, idx1, w1 = level(skip1, skip1.shape[2])
    return _tri_call(t0, t1, idx0, w0, idx1, w1)


_BLK = 6400


def _layer_body(*refs, mode, relu, nout):
    if mode == 'pre':
        x, a0, a1, w0, bb = refs[:5]
        acc = x[...] @ w0[...] + jnp.concatenate([a0[...], a1[...]], axis=1) + bb[...]
    else:
        h0, h1, a0, a1 = refs[:4]
        rest = refs[4:]
        h = jnp.concatenate([h0[...], h1[...]], axis=1)
        a = jnp.concatenate([a0[...], a1[...]], axis=1)
        if mode == 'proj':
            x, w0, w1, bb, pj = rest[:5]
        elif mode == 'id':
            r0, r1, w0, w1, bb = rest[:5]
        else:
            w0, w1, bb = rest[:3]
        acc = h @ w0[...] + a @ w1[...] + bb[...]
    if relu:
        acc = jnp.maximum(acc, 0.0)
    if mode == 'proj':
        acc = acc + x[...] @ pj[...]
    elif mode == 'id':
        acc = acc + jnp.concatenate([r0[...], r1[...]], axis=1)
    orefs = refs[-nout:]
    if nout == 2:
        orefs[0][...] = acc[:, :_CH]
        orefs[1][...] = acc[:, _CH:]
    else:
        orefs[0][...] = acc


def _row_spec(width):
    return pl.BlockSpec((_BLK, width), lambda i: (i, 0))


def _full_spec(r, ccol):
    return pl.BlockSpec((r, ccol), lambda i: (0, 0))


def _layer(mode, relu, ins, weights, d_out=32):
    nout = 2 if d_out == 32 else 1
    specs = [_row_spec(a.shape[1]) for a in ins]
    specs += [_full_spec(w.shape[0], w.shape[1]) for w in weights]
    if nout == 2:
        out_specs = [_row_spec(_CH), _row_spec(_CH)]
        out_shape = [jax.ShapeDtypeStruct((_R, _CH), jnp.float32)] * 2
    else:
        out_specs = [_row_spec(d_out)]
        out_shape = [jax.ShapeDtypeStruct((_R, d_out), jnp.float32)]
    out = pl.pallas_call(
        functools.partial(_layer_body, mode=mode, relu=relu, nout=nout),
        grid=(_R // _BLK,),
        in_specs=specs,
        out_specs=out_specs,
        out_shape=out_shape,
    )(*ins, *weights)
    return out


def _mm1_body(x_ref, w_ref, o0, o1):
    p = x_ref[...] @ w_ref[...]
    o0[...] = p[:, :_CH]
    o1[...] = p[:, _CH:]


def _mm1(x, W):
    return pl.pallas_call(
        _mm1_body,
        grid=(_R // _BLK,),
        in_specs=[_row_spec(x.shape[1]), _full_spec(W.shape[0], W.shape[1])],
        out_specs=[_row_spec(_CH), _row_spec(_CH)],
        out_shape=[jax.ShapeDtypeStruct((_R, _CH), jnp.float32)] * 2,
    )(x, W)


def kernel(t, Vt, edge_index, in_graph_features, skip0, skip1, params):
    batch_size, Vn, Dn = Vt.shape
    n = batch_size * Vn
    o0, o1 = _sample_skips(Vt, skip0, skip1)
    vt2 = jnp.zeros((_R, Dn), jnp.float32).at[:n].set(Vt.reshape(n, Dn))
    ig2 = jnp.zeros((_R, in_graph_features.shape[2]), jnp.float32
                    ).at[:n].set(in_graph_features.reshape(n, -1))
    latent = jnp.concatenate([vt2, ig2, o0, o1], axis=1)

    src = edge_index[0]
    dst = edge_index[1]
    pad = _EPAD - _E
    ar = jnp.arange(pad, dtype=jnp.int32)
    srcp = jnp.concatenate([src, ar % _N])
    dstp = jnp.concatenate([dst, _R + (ar % _PADROWS)])
    zro = jnp.zeros((_ZCH, _CH), jnp.float32)

    def spmm(p0, p1):
        return _spmm_call(p0, p1, srcp, dstp, zro)

    blocks = params['blocks']

    layers = blocks[0]['layers']
    (W0, W1, b) = layers[0]
    p0, p1 = _mm1(latent, W1)
    a0, a1 = spmm(p0, p1)
    h0, h1 = _layer('pre', True, [latent, a0, a1], [W0, b.reshape(1, -1)])
    (W0, W1, b) = layers[1]
    a0, a1 = spmm(h0, h1)
    h0, h1 = _layer('mid', True, [h0, h1, a0, a1], [W0, W1, b.reshape(1, -1)])
    (W0, W1, b) = layers[2]
    a0, a1 = spmm(h0, h1)
    x0, x1 = _layer('proj', True, [h0, h1, a0, a1, latent],
                    [W0, W1, b.reshape(1, -1), blocks[0]['proj']])

    for blk in blocks[1:]:
        layers = blk['layers']
        h0, h1 = x0, x1
        for li, (W0, W1, b) in enumerate(layers):
            a0, a1 = spmm(h0, h1)
            if li < len(layers) - 1:
                h0, h1 = _layer('mid', True, [h0, h1, a0, a1],
                                [W0, W1, b.reshape(1, -1)])
            else:
                h0, h1 = _layer('id', True, [h0, h1, a0, a1, x0, x1],
                                [W0, W1, b.reshape(1, -1)])
        x0, x1 = h0, h1

    (W0, W1, b) = params['f2v']
    a0, a1 = spmm(x0, x1)
    dV, = _layer('f2v', False, [x0, x1, a0, a1],
                 [W0, W1, b.reshape(1, -1)], d_out=Dn)
    return dV[:n].reshape(batch_size, Vn, Dn)
